# baseline (speedup 1.0000x reference)
import jax
import jax.numpy as jnp
from jax import lax
from jax.experimental import pallas as pl
from jax.experimental.pallas import tpu as pltpu
from jax.experimental.pallas import tpu_sc as plsc

_N = 10000
_E = 320000
_D = 128
_NC = 2
_NS = 16
_NW = _NC * _NS
_NP = 10240
_RPT = _NP // _NS
_EPW = _E // _NW
_CH = 80
_NIT = _EPW // _CH
_ACH = 40
_ANIT = _EPW // _ACH
_NSLOT = 6

_mesh = plsc.VectorSubcoreMesh(core_axis_name="c", subcore_axis_name="s")


_NBLK = _E // 128
_BPW = _NBLK // _NW
_BREM = _NBLK % _NW


def _sc_deg_body(ei_hbm, deg_hbm, src_hbm, dst_hbm, dacc_sh,
                 e0, e1, upd_v, buf_v, es0, es1, ws0, ws1):
    c = lax.axis_index("c")
    s = lax.axis_index("s")
    wid = c * _NS + s
    ebufs = (e0, e1)
    esems = (es0, es1)
    wsems = (ws0, ws1)
    nblk = _BPW + jnp.where(wid < _BREM, 1, 0)
    blk0 = wid * _BPW + jnp.minimum(wid, _BREM)

    def _ecopy(blk, b):
        return pltpu.async_copy(
            ei_hbm.at[:, pl.ds(blk * 128, 128)], ebufs[b], esems[b])

    def _ewait(blk, b):
        pltpu.make_async_copy(
            ei_hbm.at[:, pl.ds(blk * 128, 128)], ebufs[b], esems[b]).wait()

    _ecopy(blk0, 0)
    _ecopy(blk0 + 1, 1)

    for k in range(128 // 16):
        upd_v[pl.ds(k * 16, 16)] = jnp.ones((16,), jnp.float32)

    val = jnp.where(c == 0, 1.0, 0.0).astype(jnp.float32)

    def _fill(k, carry):
        buf_v[pl.ds(k * 16, 16)] = jnp.full((16,), 1.0, jnp.float32) * val
        return carry

    lax.fori_loop(0, _RPT // 16, _fill, 0)
    pltpu.sync_copy(buf_v, dacc_sh.at[pl.ds(s * _RPT, _RPT)])
    plsc.subcore_barrier()

    def _one(i, b):
        blk = blk0 + i
        _ewait(blk, b)
        wr = pltpu.async_copy(
            ebufs[b].at[0], src_hbm.at[pl.ds(blk * 128, 128)], wsems[b])
        pltpu.sync_copy(upd_v, dacc_sh.at[ebufs[b].at[1]], add=True)
        wr.wait()
        pltpu.sync_copy(ebufs[b].at[1],
                        dst_hbm.at[pl.ds(blk * 128, 128)])

        @pl.when(i + 2 < nblk)
        def _():
            _ecopy(blk + 2, b)

    def _pair(k, carry):
        _one(2 * k, 0)
        _one(2 * k + 1, 1)
        return carry

    lax.fori_loop(0, _BPW // 2, _pair, 0)

    @pl.when(nblk > _BPW)
    def _():
        _one(_BPW, 0)

    plsc.subcore_barrier()

    pltpu.sync_copy(dacc_sh.at[pl.ds(s * _RPT, _RPT)],
                    deg_hbm.at[c, pl.ds(s * _RPT, _RPT)])


_sc_deg = pl.kernel(
    _sc_deg_body,
    out_type=(
        jax.ShapeDtypeStruct((_NC, _NP), jnp.float32),
        jax.ShapeDtypeStruct((_E,), jnp.int32),
        jax.ShapeDtypeStruct((_E,), jnp.int32),
    ),
    mesh=_mesh,
    scratch_types=[
        pltpu.VMEM_SHARED((_NP,), jnp.float32),
        pltpu.VMEM((2, 128), jnp.int32),
        pltpu.VMEM((2, 128), jnp.int32),
        pltpu.VMEM((128,), jnp.float32),
        pltpu.VMEM((_RPT,), jnp.float32),
        pltpu.SemaphoreType.DMA,
        pltpu.SemaphoreType.DMA,
        pltpu.SemaphoreType.DMA,
        pltpu.SemaphoreType.DMA,
    ],
)


def _sc_agg_body(hs_hbm, src_hbm, dst_hbm, out_hbm,
                 acc_sh, sibufs, dibufs, rows, gsems, ssems, xsems, dsems):
    c = lax.axis_index("c")
    s = lax.axis_index("s")
    wid = c * _NS + s
    rb = s * _RPT
    ebase = wid * _EPW

    def _sidx_copy(chunk, b):
        return pltpu.async_copy(
            src_hbm.at[pl.ds(ebase + chunk * _ACH, _ACH)], sibufs[b],
            xsems[b])

    def _sidx_wait(chunk, b):
        pltpu.make_async_copy(
            src_hbm.at[pl.ds(ebase + chunk * _ACH, _ACH)], sibufs[b],
            xsems[b]).wait()

    def _didx_copy(chunk, b):
        return pltpu.async_copy(
            dst_hbm.at[pl.ds(ebase + chunk * _ACH, _ACH)], dibufs[b],
            dsems[b])

    def _didx_wait(chunk, b):
        pltpu.make_async_copy(
            dst_hbm.at[pl.ds(ebase + chunk * _ACH, _ACH)], dibufs[b],
            dsems[b]).wait()

    for b in range(_NSLOT):
        _sidx_copy(b, b)
        _didx_copy(b, b)

    @pl.when(c == 0)
    def _():
        pltpu.sync_copy(hs_hbm.at[pl.ds(rb, _RPT)],
                        acc_sh.at[pl.ds(rb, _RPT)])

    @pl.when(c == 1)
    def _():
        def _zrow(r, carry):
            for k in range(_D // 16):
                rows[0][r, pl.ds(k * 16, 16)] = jnp.zeros((16,), jnp.float32)
            return carry

        lax.fori_loop(0, _ACH, _zrow, 0)

        def _zcp(j, carry):
            pltpu.sync_copy(rows[0], acc_sh.at[pl.ds(rb + j * _ACH, _ACH)])
            return carry

        lax.fori_loop(0, _RPT // _ACH, _zcp, 0)

    plsc.subcore_barrier()

    def _gather(chunk, b):
        _sidx_wait(chunk, b)
        return pltpu.async_copy(hs_hbm.at[sibufs[b]], rows[b], gsems[b])

    def _gather_wait(chunk, b):
        pltpu.make_async_copy(hs_hbm.at[sibufs[b]], rows[b],
                              gsems[b]).wait()

    for b in range(_NSLOT):
        _gather(b, b)

    _NGRP = _ANIT // _NSLOT

    def _group(k, carry):
        g = k * _NSLOT
        scats = []
        for b in range(_NSLOT):
            _gather_wait(g + b, b)

            @pl.when(g + b + _NSLOT < _ANIT)
            def _():
                _sidx_copy(g + b + _NSLOT, b)

            _didx_wait(g + b, b)
            scats.append(
                pltpu.async_copy(rows[b], acc_sh.at[dibufs[b]],
                                 ssems[b], add=True))
        for b in range(_NSLOT):
            scats[b].wait()

            @pl.when(g + b + _NSLOT < _ANIT)
            def _():
                _didx_copy(g + b + _NSLOT, b)
                _gather(g + b + _NSLOT, b)

        return carry

    lax.fori_loop(0, _NGRP, _group, 0)
    for t in range(_ANIT - _NGRP * _NSLOT):
        ct = _NGRP * _NSLOT + t
        _gather_wait(ct, t)
        _didx_wait(ct, t)
        pltpu.sync_copy(rows[t], acc_sh.at[dibufs[t]], add=True)
    plsc.subcore_barrier()

    pltpu.sync_copy(acc_sh.at[pl.ds(rb, _RPT)],
                    out_hbm.at[c, pl.ds(rb, _RPT)])


def _sc_agg_entry(hs_hbm, src_hbm, dst_hbm, out_hbm, acc_sh,
                  *rest):
    sibufs = rest[:_NSLOT]
    dibufs = rest[_NSLOT:2 * _NSLOT]
    rows = rest[2 * _NSLOT:3 * _NSLOT]
    gsems = rest[3 * _NSLOT:4 * _NSLOT]
    ssems = rest[4 * _NSLOT:5 * _NSLOT]
    xsems = rest[5 * _NSLOT:6 * _NSLOT]
    dsems = rest[6 * _NSLOT:7 * _NSLOT]
    _sc_agg_body(hs_hbm, src_hbm, dst_hbm, out_hbm, acc_sh,
                 sibufs, dibufs, rows, gsems, ssems, xsems, dsems)


_sc_agg = pl.kernel(
    _sc_agg_entry,
    out_type=jax.ShapeDtypeStruct((_NC, _NP, _D), jnp.float32),
    mesh=_mesh,
    scratch_types=(
        [pltpu.VMEM_SHARED((_NP, _D), jnp.float32)]
        + [pltpu.VMEM((_ACH,), jnp.int32)] * (2 * _NSLOT)
        + [pltpu.VMEM((_ACH, _D), jnp.float32)] * _NSLOT
        + [pltpu.SemaphoreType.DMA] * (4 * _NSLOT)
    ),
)


def _tc_rsqrt_body(d_ref, o_ref):
    o_ref[...] = lax.rsqrt(d_ref[0] + d_ref[1])


def _tc_rsqrt(deg2):
    return pl.pallas_call(
        _tc_rsqrt_body,
        out_shape=jax.ShapeDtypeStruct((_NP // 128, 128), jnp.float32),
    )(deg2)


def _tc_mm1_body(x_ref, w_ref, dv_ref, o_ref):
    o_ref[_N :, :] = jnp.zeros((_NP - _N, _D), jnp.float32)
    o_ref[: _N, :] = jnp.dot(
        x_ref[...], w_ref[...], preferred_element_type=jnp.float32
    ) * dv_ref[...]


def _tc_mm1(x, W, dinv_col):
    return pl.pallas_call(
        _tc_mm1_body,
        out_shape=jax.ShapeDtypeStruct((_NP, _D), jnp.float32),
    )(x, W, dinv_col)


def _tc_mid_body(p_ref, dv_ref, b_ref, w_ref, o_ref):
    agg = p_ref[0, : _N, :] + p_ref[1, : _N, :]
    h = agg * dv_ref[...] + b_ref[...]
    h = jnp.maximum(h, 0.0)
    o_ref[_N :, :] = jnp.zeros((_NP - _N, _D), jnp.float32)
    o_ref[: _N, :] = jnp.dot(
        h, w_ref[...], preferred_element_type=jnp.float32
    ) * dv_ref[...]


def _tc_mid(p, dinv_col, b1, W2):
    return pl.pallas_call(
        _tc_mid_body,
        out_shape=jax.ShapeDtypeStruct((_NP, _D), jnp.float32),
    )(p, dinv_col, b1, W2)


def _tc_out_body(q_ref, dv_ref, b_ref, o_ref):
    agg = q_ref[0, : _N, :] + q_ref[1, : _N, :]
    o_ref[...] = agg * dv_ref[...] + b_ref[...]


def _tc_out(q, dinv_col, b2):
    return pl.pallas_call(
        _tc_out_body,
        out_shape=jax.ShapeDtypeStruct((_N, _D), jnp.float32),
    )(q, dinv_col, b2)


def kernel(x, edge_index, W1, b1, W2, b2):
    deg2, src, dst = _sc_deg(edge_index)
    dinvp = _tc_rsqrt(deg2.reshape(_NC, _NP // 128, 128))
    dinv_col = dinvp.reshape(_NP, 1)[:_N]

    hs1 = _tc_mm1(x, W1, dinv_col)
    p = _sc_agg(hs1, src, dst)
    hs2 = _tc_mid(p, dinv_col, b1.reshape(1, _D), W2)

    q = _sc_agg(hs2, src, dst)
    return _tc_out(q, dinv_col, b2.reshape(1, _D))


# speedup vs baseline: 1.0441x; 1.0441x over previous
import jax
import jax.numpy as jnp
from jax import lax
from jax.experimental import pallas as pl
from jax.experimental.pallas import tpu as pltpu
from jax.experimental.pallas import tpu_sc as plsc

_N = 10000
_E = 320000
_D = 128
_NC = 2
_NS = 16
_NW = _NC * _NS
_NP = 10240
_RPT = _NP // _NS
_EPW = _E // _NW
_CH = 80
_NIT = _EPW // _CH
_ACH = 40
_ANIT = _EPW // _ACH
_NSLOT = 6

_mesh = plsc.VectorSubcoreMesh(core_axis_name="c", subcore_axis_name="s")


_NBLK = _E // 128
_BPW = _NBLK // _NW
_BREM = _NBLK % _NW


def _sc_deg_body(ei_hbm, deg_hbm, dst_hbm, dacc_sh,
                 e0, e1, e2, e3, upd_v, buf_v, *sems):
    c = lax.axis_index("c")
    s = lax.axis_index("s")
    wid = c * _NS + s
    ebufs = (e0, e1, e2, e3)
    esems = sems[0:4]
    wsems = sems[4:8]
    csems = sems[8:12]
    nblk = _BPW + jnp.where(wid < _BREM, 1, 0)
    blk0 = wid * _BPW + jnp.minimum(wid, _BREM)

    def _ecopy(blk, b):
        return pltpu.async_copy(
            ei_hbm.at[:, pl.ds(blk * 128, 128)], ebufs[b], esems[b])

    def _ewait(blk, b):
        pltpu.make_async_copy(
            ei_hbm.at[:, pl.ds(blk * 128, 128)], ebufs[b], esems[b]).wait()

    for b in range(4):
        _ecopy(blk0 + b, b)

    for k in range(128 // 16):
        upd_v[pl.ds(k * 16, 16)] = jnp.ones((16,), jnp.float32)

    val = jnp.where(c == 0, 1.0, 0.0).astype(jnp.float32)

    def _fill(k, carry):
        buf_v[pl.ds(k * 16, 16)] = jnp.full((16,), 1.0, jnp.float32) * val
        return carry

    lax.fori_loop(0, _RPT // 16, _fill, 0)
    pltpu.sync_copy(buf_v, dacc_sh.at[pl.ds(s * _RPT, _RPT)])
    plsc.subcore_barrier()

    def _start(i, b):
        blk = blk0 + i
        _ewait(blk, b)
        w = pltpu.async_copy(
            ebufs[b].at[1], dst_hbm.at[pl.ds(blk * 128, 128)], wsems[b])
        sc = pltpu.async_copy(upd_v, dacc_sh.at[ebufs[b].at[1]], csems[b],
                              add=True)
        return w, sc

    def _quad(k, carry):
        g = 4 * k
        ds_ = [_start(g + b, b) for b in range(4)]
        for b in range(4):
            ds_[b][0].wait()
            ds_[b][1].wait()

            @pl.when(g + b + 4 < nblk)
            def _():
                _ecopy(blk0 + g + b + 4, b)

        return carry

    lax.fori_loop(0, _BPW // 4, _quad, 0)
    for t in range(_BPW - _BPW // 4 * 4 + 1):
        i = _BPW // 4 * 4 + t

        @pl.when(i < nblk)
        def _():
            w, sc = _start(i, t)
            w.wait()
            sc.wait()

    plsc.subcore_barrier()

    pltpu.sync_copy(dacc_sh.at[pl.ds(s * _RPT, _RPT)],
                    deg_hbm.at[c, pl.ds(s * _RPT, _RPT)])


_sc_deg = pl.kernel(
    _sc_deg_body,
    out_type=(
        jax.ShapeDtypeStruct((_NC, _NP), jnp.float32),
        jax.ShapeDtypeStruct((_E,), jnp.int32),
    ),
    mesh=_mesh,
    scratch_types=(
        [pltpu.VMEM_SHARED((_NP,), jnp.float32)]
        + [pltpu.VMEM((2, 128), jnp.int32)] * 4
        + [pltpu.VMEM((128,), jnp.float32)]
        + [pltpu.VMEM((_RPT,), jnp.float32)]
        + [pltpu.SemaphoreType.DMA] * 12
    ),
)


def _sc_agg_body(hs_hbm, src_hbm, dst_hbm, out_hbm,
                 acc_sh, sibufs, dibufs, rows, gsems, ssems, xsems, dsems):
    c = lax.axis_index("c")
    s = lax.axis_index("s")
    wid = c * _NS + s
    rb = s * _RPT
    ebase = wid * _EPW

    def _sidx_copy(chunk, b):
        return pltpu.async_copy(
            src_hbm.at[pl.ds(ebase + chunk * _ACH, _ACH)], sibufs[b],
            xsems[b])

    def _sidx_wait(chunk, b):
        pltpu.make_async_copy(
            src_hbm.at[pl.ds(ebase + chunk * _ACH, _ACH)], sibufs[b],
            xsems[b]).wait()

    def _didx_copy(chunk, b):
        return pltpu.async_copy(
            dst_hbm.at[pl.ds(ebase + chunk * _ACH, _ACH)], dibufs[b],
            dsems[b])

    def _didx_wait(chunk, b):
        pltpu.make_async_copy(
            dst_hbm.at[pl.ds(ebase + chunk * _ACH, _ACH)], dibufs[b],
            dsems[b]).wait()

    for b in range(_NSLOT):
        _sidx_copy(b, b)
        _didx_copy(b, b)

    @pl.when(c == 0)
    def _():
        pltpu.sync_copy(hs_hbm.at[pl.ds(rb, _RPT)],
                        acc_sh.at[pl.ds(rb, _RPT)])

    @pl.when(c == 1)
    def _():
        def _zrow(r, carry):
            for k in range(_D // 16):
                rows[0][r, pl.ds(k * 16, 16)] = jnp.zeros((16,), jnp.float32)
            return carry

        lax.fori_loop(0, _ACH, _zrow, 0)

        def _zcp(j, carry):
            pltpu.sync_copy(rows[0], acc_sh.at[pl.ds(rb + j * _ACH, _ACH)])
            return carry

        lax.fori_loop(0, _RPT // _ACH, _zcp, 0)

    plsc.subcore_barrier()

    def _gather(chunk, b):
        _sidx_wait(chunk, b)
        return pltpu.async_copy(hs_hbm.at[sibufs[b]], rows[b], gsems[b])

    def _gather_wait(chunk, b):
        pltpu.make_async_copy(hs_hbm.at[sibufs[b]], rows[b],
                              gsems[b]).wait()

    for b in range(_NSLOT):
        _gather(b, b)

    _NGRP = _ANIT // _NSLOT

    def _group(k, carry):
        g = k * _NSLOT
        scats = []
        for b in range(_NSLOT):
            _gather_wait(g + b, b)

            @pl.when(g + b + _NSLOT < _ANIT)
            def _():
                _sidx_copy(g + b + _NSLOT, b)

            _didx_wait(g + b, b)
            scats.append(
                pltpu.async_copy(rows[b], acc_sh.at[dibufs[b]],
                                 ssems[b], add=True))
        for b in range(_NSLOT):
            scats[b].wait()

            @pl.when(g + b + _NSLOT < _ANIT)
            def _():
                _didx_copy(g + b + _NSLOT, b)
                _gather(g + b + _NSLOT, b)

        return carry

    lax.fori_loop(0, _NGRP, _group, 0)
    for t in range(_ANIT - _NGRP * _NSLOT):
        ct = _NGRP * _NSLOT + t
        _gather_wait(ct, t)
        _didx_wait(ct, t)
        pltpu.sync_copy(rows[t], acc_sh.at[dibufs[t]], add=True)
    plsc.subcore_barrier()

    pltpu.sync_copy(acc_sh.at[pl.ds(rb, _RPT)],
                    out_hbm.at[c, pl.ds(rb, _RPT)])


def _sc_agg_entry(hs_hbm, src_hbm, dst_hbm, out_hbm, acc_sh,
                  *rest):
    sibufs = rest[:_NSLOT]
    dibufs = rest[_NSLOT:2 * _NSLOT]
    rows = rest[2 * _NSLOT:3 * _NSLOT]
    gsems = rest[3 * _NSLOT:4 * _NSLOT]
    ssems = rest[4 * _NSLOT:5 * _NSLOT]
    xsems = rest[5 * _NSLOT:6 * _NSLOT]
    dsems = rest[6 * _NSLOT:7 * _NSLOT]
    _sc_agg_body(hs_hbm, src_hbm, dst_hbm, out_hbm, acc_sh,
                 sibufs, dibufs, rows, gsems, ssems, xsems, dsems)


_sc_agg = pl.kernel(
    _sc_agg_entry,
    out_type=jax.ShapeDtypeStruct((_NC, _NP, _D), jnp.float32),
    mesh=_mesh,
    scratch_types=(
        [pltpu.VMEM_SHARED((_NP, _D), jnp.float32)]
        + [pltpu.VMEM((_ACH,), jnp.int32)] * (2 * _NSLOT)
        + [pltpu.VMEM((_ACH, _D), jnp.float32)] * _NSLOT
        + [pltpu.SemaphoreType.DMA] * (4 * _NSLOT)
    ),
)


def _tc_rsqrt_body(d_ref, o_ref):
    o_ref[...] = lax.rsqrt(d_ref[0] + d_ref[1])


def _tc_rsqrt(deg2):
    return pl.pallas_call(
        _tc_rsqrt_body,
        out_shape=jax.ShapeDtypeStruct((_NP // 128, 128), jnp.float32),
    )(deg2)


def _tc_mm1_body(x_ref, w_ref, dv_ref, o_ref):
    o_ref[_N :, :] = jnp.zeros((_NP - _N, _D), jnp.float32)
    o_ref[: _N, :] = jnp.dot(
        x_ref[...], w_ref[...], preferred_element_type=jnp.float32
    ) * dv_ref[...]


def _tc_mm1(x, W, dinv_col):
    return pl.pallas_call(
        _tc_mm1_body,
        out_shape=jax.ShapeDtypeStruct((_NP, _D), jnp.float32),
    )(x, W, dinv_col)


def _tc_mid_body(p_ref, dv_ref, b_ref, w_ref, o_ref):
    agg = p_ref[0, : _N, :] + p_ref[1, : _N, :]
    h = agg * dv_ref[...] + b_ref[...]
    h = jnp.maximum(h, 0.0)
    o_ref[_N :, :] = jnp.zeros((_NP - _N, _D), jnp.float32)
    o_ref[: _N, :] = jnp.dot(
        h, w_ref[...], preferred_element_type=jnp.float32
    ) * dv_ref[...]


def _tc_mid(p, dinv_col, b1, W2):
    return pl.pallas_call(
        _tc_mid_body,
        out_shape=jax.ShapeDtypeStruct((_NP, _D), jnp.float32),
    )(p, dinv_col, b1, W2)


def _tc_out_body(q_ref, dv_ref, b_ref, o_ref):
    agg = q_ref[0, : _N, :] + q_ref[1, : _N, :]
    o_ref[...] = agg * dv_ref[...] + b_ref[...]


def _tc_out(q, dinv_col, b2):
    return pl.pallas_call(
        _tc_out_body,
        out_shape=jax.ShapeDtypeStruct((_N, _D), jnp.float32),
    )(q, dinv_col, b2)


def kernel(x, edge_index, W1, b1, W2, b2):
    deg2, dst = _sc_deg(edge_index)
    src = edge_index[0]
    dinvp = _tc_rsqrt(deg2.reshape(_NC, _NP // 128, 128))
    dinv_col = dinvp.reshape(_NP, 1)[:_N]

    hs1 = _tc_mm1(x, W1, dinv_col)
    p = _sc_agg(hs1, src, dst)
    hs2 = _tc_mid(p, dinv_col, b1.reshape(1, _D), W2)

    q = _sc_agg(hs2, src, dst)
    return _tc_out(q, dinv_col, b2.reshape(1, _D))
